# baseline (speedup 1.0000x reference)
import functools

import jax
import jax.numpy as jnp
from jax import lax
from jax.experimental import pallas as pl
from jax.experimental.pallas import tpu as pltpu
from jax.experimental.pallas import tpu_sc as plsc

VOCAB = 100000
DIM = 100
BATCH = 16384
N_ROWS = 2 * BATCH

_info = plsc.get_sparse_core_info()
_NC, _NS = _info.num_cores, _info.num_subcores
_NW = _NC * _NS
_B_PER_W = N_ROWS // _NW

_mesh = plsc.VectorSubcoreMesh(core_axis_name="c", subcore_axis_name="s")

_CHUNK = 128
_N_CHUNKS = _B_PER_W // _CHUNK


_DPAD = 128


@functools.partial(
    pl.kernel,
    mesh=_mesh,
    out_type=jax.ShapeDtypeStruct((N_ROWS, _DPAD), jnp.float32),
    scratch_types=[
        pltpu.VMEM((_N_CHUNKS, _CHUNK), jnp.int32),
        pltpu.VMEM((_CHUNK, _DPAD), jnp.float32),
        pltpu.SemaphoreType.DMA,
    ],
)
def _gather_rows(table_hbm, idx_hbm, out_hbm, idx_v, rows_v, sem):
    wid = lax.axis_index("s") * _NC + lax.axis_index("c")
    base = wid * _B_PER_W
    pltpu.sync_copy(idx_hbm.at[pl.ds(wid * _N_CHUNKS, _N_CHUNKS)], idx_v)
    for j in range(_N_CHUNKS):
        pltpu.async_copy(table_hbm.at[idx_v.at[j]], rows_v, sem).wait()
        pltpu.sync_copy(rows_v, out_hbm.at[pl.ds(base + j * _CHUNK, _CHUNK)])


def kernel(word_embs, obj_category, sub_category):
    idx = jnp.stack(
        [obj_category.astype(jnp.int32), sub_category.astype(jnp.int32)], axis=1
    ).reshape(N_ROWS // _CHUNK, _CHUNK)
    table_p = jnp.pad(word_embs, ((0, 0), (0, _DPAD - DIM)))
    rows = _gather_rows(table_p, idx).reshape(BATCH, 2 * _DPAD)
    return jnp.concatenate([rows[:, :DIM], rows[:, _DPAD : _DPAD + DIM]], axis=1)


# speedup vs baseline: 1.1201x; 1.1201x over previous
import functools

import jax
import jax.numpy as jnp
from jax import lax
from jax.experimental import pallas as pl
from jax.experimental.pallas import tpu as pltpu
from jax.experimental.pallas import tpu_sc as plsc

VOCAB = 100000
DIM = 100
BATCH = 16384
N_ROWS = 2 * BATCH
_DPAD = 128

_info = plsc.get_sparse_core_info()
_NC, _NS = _info.num_cores, _info.num_subcores
_NW = _NC * _NS
_B_PER_W = N_ROWS // _NW

_mesh = plsc.VectorSubcoreMesh(core_axis_name="c", subcore_axis_name="s")

_CHUNK = 128
_N_CHUNKS = _B_PER_W // _CHUNK


@functools.partial(
    pl.kernel,
    mesh=_mesh,
    out_type=jax.ShapeDtypeStruct((N_ROWS, _DPAD), jnp.float32),
    scratch_types=[
        pltpu.VMEM((_N_CHUNKS, _CHUNK), jnp.int32),
        pltpu.VMEM((_CHUNK, _DPAD), jnp.float32),
        pltpu.SemaphoreType.DMA,
    ],
)
def _gather_rows(table_hbm, idx_hbm, out_hbm, idx_v, rows_v, sem):
    wid = lax.axis_index("s") * _NC + lax.axis_index("c")
    base = wid * _B_PER_W
    pltpu.sync_copy(idx_hbm.at[pl.ds(wid * _N_CHUNKS, _N_CHUNKS)], idx_v)
    for j in range(_N_CHUNKS):
        pltpu.async_copy(table_hbm.at[idx_v.at[j]], rows_v, sem).wait()
        pltpu.sync_copy(rows_v, out_hbm.at[pl.ds(base + j * _CHUNK, _CHUNK)])


_PAD_ROWS = 400


def _pad_body(x_ref, o_ref):
    o_ref[...] = jnp.concatenate(
        [x_ref[...], jnp.zeros((_PAD_ROWS, _DPAD - DIM), jnp.float32)], axis=1
    )


_pad_table = pl.pallas_call(
    _pad_body,
    grid=(VOCAB // _PAD_ROWS,),
    in_specs=[pl.BlockSpec((_PAD_ROWS, DIM), lambda i: (i, 0))],
    out_specs=pl.BlockSpec((_PAD_ROWS, _DPAD), lambda i: (i, 0)),
    out_shape=jax.ShapeDtypeStruct((VOCAB, _DPAD), jnp.float32),
)

_CAT_ROWS = 512


def _concat_body(obj_ref, sub_ref, o_ref):
    o_ref[...] = jnp.concatenate(
        [obj_ref[:, :DIM], sub_ref[:, :DIM]], axis=1
    )


_concat_out = pl.pallas_call(
    _concat_body,
    grid=(BATCH // _CAT_ROWS,),
    in_specs=[
        pl.BlockSpec((_CAT_ROWS, _DPAD), lambda i: (i, 0)),
        pl.BlockSpec((_CAT_ROWS, _DPAD), lambda i: (i + BATCH // _CAT_ROWS, 0)),
    ],
    out_specs=pl.BlockSpec((_CAT_ROWS, 2 * DIM), lambda i: (i, 0)),
    out_shape=jax.ShapeDtypeStruct((BATCH, 2 * DIM), jnp.float32),
)


def kernel(word_embs, obj_category, sub_category):
    idx = jnp.concatenate(
        [obj_category.astype(jnp.int32), sub_category.astype(jnp.int32)]
    ).reshape(N_ROWS // _CHUNK, _CHUNK)
    table_p = _pad_table(word_embs)
    rows = _gather_rows(table_p, idx)
    return _concat_out(rows, rows)


# speedup vs baseline: 1.9181x; 1.7125x over previous
import functools

import jax
import jax.numpy as jnp
from jax import lax
from jax.experimental import pallas as pl
from jax.experimental.pallas import tpu as pltpu
from jax.experimental.pallas import tpu_sc as plsc

VOCAB = 100000
DIM = 100
BATCH = 16384
N_ROWS = 2 * BATCH
_DPAD = 128

_info = plsc.get_sparse_core_info()
_NC, _NS = _info.num_cores, _info.num_subcores
_NW = _NC * _NS
_B_PER_W = N_ROWS // _NW

_mesh = plsc.VectorSubcoreMesh(core_axis_name="c", subcore_axis_name="s")

_CHUNK = 128
_N_CHUNKS = _B_PER_W // _CHUNK


@functools.partial(
    pl.kernel,
    mesh=_mesh,
    out_type=jax.ShapeDtypeStruct((N_ROWS, _DPAD), jnp.float32),
    scratch_types=[
        pltpu.VMEM((_N_CHUNKS, _CHUNK), jnp.int32),
        pltpu.VMEM((_CHUNK, _DPAD), jnp.float32),
        pltpu.SemaphoreType.DMA,
    ],
)
def _gather_rows(table_hbm, idx_hbm, out_hbm, idx_v, rows_v, sem):
    wid = lax.axis_index("s") * _NC + lax.axis_index("c")
    base = wid * _B_PER_W
    pltpu.sync_copy(idx_hbm.at[pl.ds(wid * _N_CHUNKS, _N_CHUNKS)], idx_v)
    for j in range(_N_CHUNKS):
        pltpu.async_copy(table_hbm.at[idx_v.at[j]], rows_v, sem).wait()
        pltpu.sync_copy(rows_v, out_hbm.at[pl.ds(base + j * _CHUNK, _CHUNK)])


_PAD_ROWS = 4000


def _pad_body(x_ref, o_ref):
    o_ref[:, :DIM] = x_ref[...]


_pad_table = pl.pallas_call(
    _pad_body,
    grid=(VOCAB // _PAD_ROWS,),
    in_specs=[pl.BlockSpec((_PAD_ROWS, DIM), lambda i: (i, 0))],
    out_specs=pl.BlockSpec((_PAD_ROWS, _DPAD), lambda i: (i, 0)),
    out_shape=jax.ShapeDtypeStruct((VOCAB, _DPAD), jnp.float32),
)

_CAT_ROWS = 512


def _concat_body(obj_ref, sub_ref, o_ref):
    o_ref[...] = jnp.concatenate(
        [obj_ref[:, :DIM], sub_ref[:, :DIM]], axis=1
    )


_concat_out = pl.pallas_call(
    _concat_body,
    grid=(BATCH // _CAT_ROWS,),
    in_specs=[
        pl.BlockSpec((_CAT_ROWS, _DPAD), lambda i: (i, 0)),
        pl.BlockSpec((_CAT_ROWS, _DPAD), lambda i: (i + BATCH // _CAT_ROWS, 0)),
    ],
    out_specs=pl.BlockSpec((_CAT_ROWS, 2 * DIM), lambda i: (i, 0)),
    out_shape=jax.ShapeDtypeStruct((BATCH, 2 * DIM), jnp.float32),
)


def kernel(word_embs, obj_category, sub_category):
    idx = jnp.concatenate(
        [obj_category.astype(jnp.int32), sub_category.astype(jnp.int32)]
    ).reshape(N_ROWS // _CHUNK, _CHUNK)
    table_p = _pad_table(word_embs)
    rows = _gather_rows(table_p, idx)
    return _concat_out(rows, rows)


# speedup vs baseline: 2.3783x; 1.2399x over previous
import functools

import jax
import jax.numpy as jnp
from jax import lax
from jax.experimental import pallas as pl
from jax.experimental.pallas import tpu as pltpu
from jax.experimental.pallas import tpu_sc as plsc

VOCAB = 100000
DIM = 100
BATCH = 16384
N_ROWS = 2 * BATCH
_DPAD = 128

_info = plsc.get_sparse_core_info()
_NC, _NS = _info.num_cores, _info.num_subcores
_NW = _NC * _NS
_B_PER_W = N_ROWS // _NW

_mesh = plsc.VectorSubcoreMesh(core_axis_name="c", subcore_axis_name="s")

_CHUNK = 128
_N_CHUNKS = _B_PER_W // _CHUNK


@functools.partial(
    pl.kernel,
    mesh=_mesh,
    out_type=jax.ShapeDtypeStruct((N_ROWS, _DPAD), jnp.float32),
    scratch_types=[
        pltpu.VMEM((_N_CHUNKS, _CHUNK), jnp.int32),
        pltpu.VMEM((_CHUNK, _DPAD), jnp.float32),
        pltpu.SemaphoreType.DMA,
    ],
)
def _gather_rows(table_hbm, idx_hbm, out_hbm, idx_v, rows_v, sem):
    wid = lax.axis_index("s") * _NC + lax.axis_index("c")
    base = wid * _B_PER_W
    pltpu.sync_copy(idx_hbm.at[pl.ds(wid * _N_CHUNKS, _N_CHUNKS)], idx_v)
    for j in range(_N_CHUNKS):
        pltpu.async_copy(table_hbm.at[idx_v.at[j]], rows_v, sem).wait()
        pltpu.sync_copy(rows_v, out_hbm.at[pl.ds(base + j * _CHUNK, _CHUNK)])


_PAD_ROWS = 2560


def _pad_body(xt_ref, o_ref):
    o_ref[:, :DIM] = jnp.transpose(xt_ref[...])


_pad_table = pl.pallas_call(
    _pad_body,
    grid=(pl.cdiv(VOCAB, _PAD_ROWS),),
    in_specs=[pl.BlockSpec((DIM, _PAD_ROWS), lambda i: (0, i))],
    out_specs=pl.BlockSpec((_PAD_ROWS, _DPAD), lambda i: (i, 0)),
    out_shape=jax.ShapeDtypeStruct((VOCAB, _DPAD), jnp.float32),
)

_CAT_ROWS = 512


def _concat_body(obj_ref, sub_ref, o_ref):
    o_ref[...] = jnp.concatenate(
        [obj_ref[:, :DIM], sub_ref[:, :DIM]], axis=1
    )


_concat_out = pl.pallas_call(
    _concat_body,
    grid=(BATCH // _CAT_ROWS,),
    in_specs=[
        pl.BlockSpec((_CAT_ROWS, _DPAD), lambda i: (i, 0)),
        pl.BlockSpec((_CAT_ROWS, _DPAD), lambda i: (i + BATCH // _CAT_ROWS, 0)),
    ],
    out_specs=pl.BlockSpec((_CAT_ROWS, 2 * DIM), lambda i: (i, 0)),
    out_shape=jax.ShapeDtypeStruct((BATCH, 2 * DIM), jnp.float32),
)


def kernel(word_embs, obj_category, sub_category):
    idx = jnp.concatenate(
        [obj_category.astype(jnp.int32), sub_category.astype(jnp.int32)]
    ).reshape(N_ROWS // _CHUNK, _CHUNK)
    table_p = _pad_table(word_embs.T)
    rows = _gather_rows(table_p, idx)
    return _concat_out(rows, rows)


# speedup vs baseline: 2.7101x; 1.1395x over previous
import functools

import jax
import jax.numpy as jnp
from jax import lax
from jax.experimental import pallas as pl
from jax.experimental.pallas import tpu as pltpu
from jax.experimental.pallas import tpu_sc as plsc

VOCAB = 100000
DIM = 100
BATCH = 16384
N_ROWS = 2 * BATCH
_DPAD = 128

_info = plsc.get_sparse_core_info()
_NC, _NS = _info.num_cores, _info.num_subcores
_NW = _NC * _NS
_B_PER_W = N_ROWS // _NW

_mesh = plsc.VectorSubcoreMesh(core_axis_name="c", subcore_axis_name="s")

_CHUNK = 128
_N_CHUNKS = _B_PER_W // _CHUNK


@functools.partial(
    pl.kernel,
    mesh=_mesh,
    out_type=jax.ShapeDtypeStruct((N_ROWS, _DPAD), jnp.float32),
    scratch_types=[
        pltpu.VMEM((_N_CHUNKS, _CHUNK), jnp.int32),
        pltpu.VMEM((_CHUNK, _DPAD), jnp.float32),
        pltpu.SemaphoreType.DMA,
    ],
)
def _gather_rows(table_hbm, idx_hbm, out_hbm, idx_v, rows_v, sem):
    wid = lax.axis_index("s") * _NC + lax.axis_index("c")
    base = wid * _B_PER_W
    pltpu.sync_copy(idx_hbm.at[pl.ds(wid * _N_CHUNKS, _N_CHUNKS)], idx_v)
    for j in range(_N_CHUNKS):
        pltpu.async_copy(table_hbm.at[idx_v.at[j]], rows_v, sem).wait()
        pltpu.sync_copy(rows_v, out_hbm.at[pl.ds(base + j * _CHUNK, _CHUNK)])


_PAD_ROWS = 2560


def _pad_body(xt_ref, o_ref):
    o_ref[:, :DIM] = jnp.transpose(xt_ref[...])


_pad_table = pl.pallas_call(
    _pad_body,
    grid=(pl.cdiv(VOCAB, _PAD_ROWS),),
    in_specs=[pl.BlockSpec((DIM, _PAD_ROWS), lambda i: (0, i))],
    out_specs=pl.BlockSpec((_PAD_ROWS, _DPAD), lambda i: (i, 0)),
    out_shape=jax.ShapeDtypeStruct((VOCAB, _DPAD), jnp.float32),
)

_CAT_ROWS = 512


def _concat_body(obj_ref, sub_ref, o_ref):
    o_ref[:DIM, :] = jnp.transpose(obj_ref[...])[:DIM, :]
    o_ref[DIM:, :] = jnp.transpose(sub_ref[...])[:DIM, :]


_concat_out = pl.pallas_call(
    _concat_body,
    grid=(BATCH // _CAT_ROWS,),
    in_specs=[
        pl.BlockSpec((_CAT_ROWS, _DPAD), lambda i: (i, 0)),
        pl.BlockSpec((_CAT_ROWS, _DPAD), lambda i: (i + BATCH // _CAT_ROWS, 0)),
    ],
    out_specs=pl.BlockSpec((2 * DIM, _CAT_ROWS), lambda i: (0, i)),
    out_shape=jax.ShapeDtypeStruct((2 * DIM, BATCH), jnp.float32),
)


def kernel(word_embs, obj_category, sub_category):
    idx = jnp.concatenate(
        [obj_category.astype(jnp.int32), sub_category.astype(jnp.int32)]
    ).reshape(N_ROWS // _CHUNK, _CHUNK)
    table_p = _pad_table(word_embs.T)
    rows = _gather_rows(table_p, idx)
    return _concat_out(rows, rows).T


# speedup vs baseline: 3.1000x; 1.1439x over previous
import functools

import jax
import jax.numpy as jnp
from jax import lax
from jax.experimental import pallas as pl
from jax.experimental.pallas import tpu as pltpu
from jax.experimental.pallas import tpu_sc as plsc

VOCAB = 100000
DIM = 100
BATCH = 16384
N_ROWS = 2 * BATCH
_DPAD = 128

_info = plsc.get_sparse_core_info()
_NC, _NS = _info.num_cores, _info.num_subcores
_NW = _NC * _NS
_B_PER_W = N_ROWS // _NW

_mesh = plsc.VectorSubcoreMesh(core_axis_name="c", subcore_axis_name="s")

_CHUNK = 128
_N_CHUNKS = _B_PER_W // _CHUNK


@functools.partial(
    pl.kernel,
    mesh=_mesh,
    out_type=jax.ShapeDtypeStruct((N_ROWS, _DPAD), jnp.float32),
    scratch_types=[
        pltpu.VMEM((_N_CHUNKS, _CHUNK), jnp.int32),
        pltpu.VMEM((2, _CHUNK, _DPAD), jnp.float32),
        pltpu.SemaphoreType.DMA,
        pltpu.SemaphoreType.DMA,
    ],
)
def _gather_rows(table_hbm, idx_hbm, out_hbm, idx_v, rows_v, gsem, osem):
    wid = lax.axis_index("s") * _NC + lax.axis_index("c")
    base = wid * _B_PER_W
    pltpu.sync_copy(idx_hbm.at[pl.ds(wid * _N_CHUNKS, _N_CHUNKS)], idx_v)
    pltpu.async_copy(table_hbm.at[idx_v.at[0]], rows_v.at[0], gsem)
    for j in range(_N_CHUNKS):
        b = j % 2
        pltpu.make_async_copy(
            table_hbm.at[idx_v.at[j]], rows_v.at[b], gsem
        ).wait()
        if j + 1 < _N_CHUNKS:
            if j >= 1:
                pltpu.make_async_copy(
                    rows_v.at[1 - b],
                    out_hbm.at[pl.ds(base + (j - 1) * _CHUNK, _CHUNK)],
                    osem,
                ).wait()
            pltpu.async_copy(
                table_hbm.at[idx_v.at[j + 1]], rows_v.at[1 - b], gsem
            )
        pltpu.async_copy(
            rows_v.at[b], out_hbm.at[pl.ds(base + j * _CHUNK, _CHUNK)], osem
        )
    for j in (_N_CHUNKS - 2, _N_CHUNKS - 1):
        pltpu.make_async_copy(
            rows_v.at[j % 2], out_hbm.at[pl.ds(base + j * _CHUNK, _CHUNK)], osem
        ).wait()


_PAD_ROWS = 5120


def _pad_body(xt_ref, o_ref):
    o_ref[:, :DIM] = jnp.transpose(xt_ref[...])


_pad_table = pl.pallas_call(
    _pad_body,
    grid=(pl.cdiv(VOCAB, _PAD_ROWS),),
    in_specs=[pl.BlockSpec((DIM, _PAD_ROWS), lambda i: (0, i))],
    out_specs=pl.BlockSpec((_PAD_ROWS, _DPAD), lambda i: (i, 0)),
    out_shape=jax.ShapeDtypeStruct((VOCAB, _DPAD), jnp.float32),
)

_CAT_ROWS = 512


def _concat_body(obj_ref, sub_ref, o_ref):
    o_ref[:DIM, :] = jnp.transpose(obj_ref[...])[:DIM, :]
    o_ref[DIM:, :] = jnp.transpose(sub_ref[...])[:DIM, :]


_concat_out = pl.pallas_call(
    _concat_body,
    grid=(BATCH // _CAT_ROWS,),
    in_specs=[
        pl.BlockSpec((_CAT_ROWS, _DPAD), lambda i: (i, 0)),
        pl.BlockSpec((_CAT_ROWS, _DPAD), lambda i: (i + BATCH // _CAT_ROWS, 0)),
    ],
    out_specs=pl.BlockSpec((2 * DIM, _CAT_ROWS), lambda i: (0, i)),
    out_shape=jax.ShapeDtypeStruct((2 * DIM, BATCH), jnp.float32),
)


def kernel(word_embs, obj_category, sub_category):
    idx = jnp.concatenate(
        [obj_category.astype(jnp.int32), sub_category.astype(jnp.int32)]
    ).reshape(N_ROWS // _CHUNK, _CHUNK)
    table_p = _pad_table(word_embs.T)
    rows = _gather_rows(table_p, idx)
    return _concat_out(rows, rows).T


# speedup vs baseline: 3.4951x; 1.1274x over previous
import functools

import jax
import jax.numpy as jnp
from jax import lax
from jax.experimental import pallas as pl
from jax.experimental.pallas import tpu as pltpu
from jax.experimental.pallas import tpu_sc as plsc

VOCAB = 100000
DIM = 100
BATCH = 16384
N_ROWS = 2 * BATCH
_DPAD = 128

_info = plsc.get_sparse_core_info()
_NC, _NS = _info.num_cores, _info.num_subcores
_NW = _NC * _NS
_B_PER_W = N_ROWS // _NW

_mesh = plsc.VectorSubcoreMesh(core_axis_name="c", subcore_axis_name="s")

_CHUNK = 128
_N_CHUNKS = _B_PER_W // _CHUNK


@functools.partial(
    pl.kernel,
    mesh=_mesh,
    out_type=jax.ShapeDtypeStruct((N_ROWS, _DPAD), jnp.float32),
    scratch_types=[
        pltpu.VMEM((_N_CHUNKS, _CHUNK), jnp.int32),
        pltpu.VMEM((2, _CHUNK, _DPAD), jnp.float32),
        pltpu.SemaphoreType.DMA,
        pltpu.SemaphoreType.DMA,
    ],
)
def _gather_rows(table_hbm, idx_hbm, out_hbm, idx_v, rows_v, gsem, osem):
    wid = lax.axis_index("s") * _NC + lax.axis_index("c")
    base = wid * _B_PER_W
    pltpu.sync_copy(idx_hbm.at[pl.ds(wid * _N_CHUNKS, _N_CHUNKS)], idx_v)
    pltpu.async_copy(table_hbm.at[idx_v.at[0]], rows_v.at[0], gsem)
    for j in range(_N_CHUNKS):
        b = j % 2
        pltpu.make_async_copy(
            table_hbm.at[idx_v.at[j]], rows_v.at[b], gsem
        ).wait()
        if j + 1 < _N_CHUNKS:
            if j >= 1:
                pltpu.make_async_copy(
                    rows_v.at[1 - b],
                    out_hbm.at[pl.ds(base + (j - 1) * _CHUNK, _CHUNK)],
                    osem,
                ).wait()
            pltpu.async_copy(
                table_hbm.at[idx_v.at[j + 1]], rows_v.at[1 - b], gsem
            )
        pltpu.async_copy(
            rows_v.at[b], out_hbm.at[pl.ds(base + j * _CHUNK, _CHUNK)], osem
        )
    for j in (_N_CHUNKS - 2, _N_CHUNKS - 1):
        pltpu.make_async_copy(
            rows_v.at[j % 2], out_hbm.at[pl.ds(base + j * _CHUNK, _CHUNK)], osem
        ).wait()


_PAD_ROWS = 10240


def _pad_body(xt_ref, o_ref):
    o_ref[:, :DIM] = jnp.transpose(xt_ref[...])


_pad_table = pl.pallas_call(
    _pad_body,
    grid=(pl.cdiv(VOCAB, _PAD_ROWS),),
    in_specs=[pl.BlockSpec((DIM, _PAD_ROWS), lambda i: (0, i))],
    out_specs=pl.BlockSpec((_PAD_ROWS, _DPAD), lambda i: (i, 0)),
    out_shape=jax.ShapeDtypeStruct((VOCAB, _DPAD), jnp.float32),
)

_CAT_ROWS = 1024


def _concat_body(obj_ref, sub_ref, o_ref):
    o_ref[:DIM, :] = jnp.transpose(obj_ref[...])[:DIM, :]
    o_ref[DIM:, :] = jnp.transpose(sub_ref[...])[:DIM, :]


_concat_out = pl.pallas_call(
    _concat_body,
    grid=(BATCH // _CAT_ROWS,),
    in_specs=[
        pl.BlockSpec((_CAT_ROWS, _DPAD), lambda i: (i, 0)),
        pl.BlockSpec((_CAT_ROWS, _DPAD), lambda i: (i + BATCH // _CAT_ROWS, 0)),
    ],
    out_specs=pl.BlockSpec((2 * DIM, _CAT_ROWS), lambda i: (0, i)),
    out_shape=jax.ShapeDtypeStruct((2 * DIM, BATCH), jnp.float32),
)


def kernel(word_embs, obj_category, sub_category):
    idx = jnp.concatenate(
        [obj_category.astype(jnp.int32), sub_category.astype(jnp.int32)]
    ).reshape(N_ROWS // _CHUNK, _CHUNK)
    table_p = _pad_table(word_embs.T)
    rows = _gather_rows(table_p, idx)
    return _concat_out(rows, rows).T


# speedup vs baseline: 3.5768x; 1.0234x over previous
import functools

import jax
import jax.numpy as jnp
from jax import lax
from jax.experimental import pallas as pl
from jax.experimental.pallas import tpu as pltpu
from jax.experimental.pallas import tpu_sc as plsc

VOCAB = 100000
DIM = 100
BATCH = 16384
HALF = BATCH // 2
N_ROWS = 2 * HALF
_DPAD = 128

_info = plsc.get_sparse_core_info()
_NC, _NS = _info.num_cores, _info.num_subcores
_NW = _NC * _NS
_B_PER_W = N_ROWS // _NW

_mesh = plsc.VectorSubcoreMesh(core_axis_name="c", subcore_axis_name="s")

_CHUNK = 128
_N_CHUNKS = _B_PER_W // _CHUNK


@functools.partial(
    pl.kernel,
    mesh=_mesh,
    out_type=jax.ShapeDtypeStruct((N_ROWS, _DPAD), jnp.float32),
    scratch_types=[
        pltpu.VMEM((_N_CHUNKS, _CHUNK), jnp.int32),
        pltpu.VMEM((2, _CHUNK, _DPAD), jnp.float32),
        pltpu.SemaphoreType.DMA,
        pltpu.SemaphoreType.DMA,
    ],
)
def _gather_rows(table_hbm, idx_hbm, out_hbm, idx_v, rows_v, gsem, osem):
    wid = lax.axis_index("s") * _NC + lax.axis_index("c")
    base = wid * _B_PER_W
    pltpu.sync_copy(idx_hbm.at[pl.ds(wid * _N_CHUNKS, _N_CHUNKS)], idx_v)
    pltpu.async_copy(table_hbm.at[idx_v.at[0]], rows_v.at[0], gsem)
    for j in range(_N_CHUNKS):
        b = j % 2
        pltpu.make_async_copy(
            table_hbm.at[idx_v.at[j]], rows_v.at[b], gsem
        ).wait()
        if j + 1 < _N_CHUNKS:
            if j >= 1:
                pltpu.make_async_copy(
                    rows_v.at[1 - b],
                    out_hbm.at[pl.ds(base + (j - 1) * _CHUNK, _CHUNK)],
                    osem,
                ).wait()
            pltpu.async_copy(
                table_hbm.at[idx_v.at[j + 1]], rows_v.at[1 - b], gsem
            )
        pltpu.async_copy(
            rows_v.at[b], out_hbm.at[pl.ds(base + j * _CHUNK, _CHUNK)], osem
        )
    for j in (_N_CHUNKS - 2, _N_CHUNKS - 1):
        pltpu.make_async_copy(
            rows_v.at[j % 2], out_hbm.at[pl.ds(base + j * _CHUNK, _CHUNK)], osem
        ).wait()


_PAD_ROWS = 10240


def _pad_body(xt_ref, o_ref):
    o_ref[:, :DIM] = jnp.transpose(xt_ref[...])


_pad_table = pl.pallas_call(
    _pad_body,
    grid=(pl.cdiv(VOCAB, _PAD_ROWS),),
    in_specs=[pl.BlockSpec((DIM, _PAD_ROWS), lambda i: (0, i))],
    out_specs=pl.BlockSpec((_PAD_ROWS, _DPAD), lambda i: (i, 0)),
    out_shape=jax.ShapeDtypeStruct((VOCAB, _DPAD), jnp.float32),
)

_CAT_ROWS = 1024
_N_CAT = HALF // _CAT_ROWS


def _concat_half_body(obj_ref, sub_ref, o_ref):
    o_ref[:DIM, :] = jnp.transpose(obj_ref[...])[:DIM, :]
    o_ref[DIM:, :] = jnp.transpose(sub_ref[...])[:DIM, :]


_concat_a = pl.pallas_call(
    _concat_half_body,
    grid=(_N_CAT,),
    in_specs=[
        pl.BlockSpec((_CAT_ROWS, _DPAD), lambda i: (i, 0)),
        pl.BlockSpec((_CAT_ROWS, _DPAD), lambda i: (i + _N_CAT, 0)),
    ],
    out_specs=pl.BlockSpec((2 * DIM, _CAT_ROWS), lambda i: (0, i)),
    out_shape=jax.ShapeDtypeStruct((2 * DIM, BATCH), jnp.float32),
)


def _concat_b_body(obj_ref, sub_ref, acc_ref, o_ref):
    del acc_ref
    o_ref[:DIM, :] = jnp.transpose(obj_ref[...])[:DIM, :]
    o_ref[DIM:, :] = jnp.transpose(sub_ref[...])[:DIM, :]


_concat_b = pl.pallas_call(
    _concat_b_body,
    grid=(_N_CAT,),
    in_specs=[
        pl.BlockSpec((_CAT_ROWS, _DPAD), lambda i: (i, 0)),
        pl.BlockSpec((_CAT_ROWS, _DPAD), lambda i: (i + _N_CAT, 0)),
        pl.BlockSpec(memory_space=pl.ANY),
    ],
    out_specs=pl.BlockSpec((2 * DIM, _CAT_ROWS), lambda i: (0, i + _N_CAT)),
    out_shape=jax.ShapeDtypeStruct((2 * DIM, BATCH), jnp.float32),
    input_output_aliases={2: 0},
)


def kernel(word_embs, obj_category, sub_category):
    obj = obj_category.astype(jnp.int32)
    sub = sub_category.astype(jnp.int32)
    idx_a = jnp.concatenate([obj[:HALF], sub[:HALF]]).reshape(
        N_ROWS // _CHUNK, _CHUNK
    )
    idx_b = jnp.concatenate([obj[HALF:], sub[HALF:]]).reshape(
        N_ROWS // _CHUNK, _CHUNK
    )
    table_p = _pad_table(word_embs.T)
    rows_a = _gather_rows(table_p, idx_a)
    rows_b = _gather_rows(table_p, idx_b)
    out_a = _concat_a(rows_a, rows_a)
    out = _concat_b(rows_b, rows_b, out_a)
    return out.T


# speedup vs baseline: 3.6970x; 1.0336x over previous
import functools

import jax
import jax.numpy as jnp
from jax import lax
from jax.experimental import pallas as pl
from jax.experimental.pallas import tpu as pltpu
from jax.experimental.pallas import tpu_sc as plsc

VOCAB = 100000
DIM = 100
BATCH = 16384
HALF = BATCH // 2
N_ROWS = 2 * HALF
_DPAD = 128

_info = plsc.get_sparse_core_info()
_NC, _NS = _info.num_cores, _info.num_subcores
_NW = _NC * _NS
_B_PER_W = N_ROWS // _NW

_mesh = plsc.VectorSubcoreMesh(core_axis_name="c", subcore_axis_name="s")

_CHUNK = 128
_N_CHUNKS = _B_PER_W // _CHUNK


@functools.partial(
    pl.kernel,
    mesh=_mesh,
    out_type=jax.ShapeDtypeStruct((N_ROWS, _DPAD), jnp.float32),
    scratch_types=[
        pltpu.VMEM((_N_CHUNKS, _CHUNK), jnp.int32),
        pltpu.VMEM((2, _CHUNK, _DPAD), jnp.float32),
        pltpu.SemaphoreType.DMA,
        pltpu.SemaphoreType.DMA,
    ],
)
def _gather_rows(table_hbm, idx_hbm, out_hbm, idx_v, rows_v, gsem, osem):
    wid = lax.axis_index("s") * _NC + lax.axis_index("c")
    base = wid * _B_PER_W
    pltpu.sync_copy(idx_hbm.at[pl.ds(wid * _N_CHUNKS, _N_CHUNKS)], idx_v)
    pltpu.async_copy(table_hbm.at[idx_v.at[0]], rows_v.at[0], gsem)
    for j in range(_N_CHUNKS):
        b = j % 2
        pltpu.make_async_copy(
            table_hbm.at[idx_v.at[j]], rows_v.at[b], gsem
        ).wait()
        if j + 1 < _N_CHUNKS:
            if j >= 1:
                pltpu.make_async_copy(
                    rows_v.at[1 - b],
                    out_hbm.at[pl.ds(base + (j - 1) * _CHUNK, _CHUNK)],
                    osem,
                ).wait()
            pltpu.async_copy(
                table_hbm.at[idx_v.at[j + 1]], rows_v.at[1 - b], gsem
            )
        pltpu.async_copy(
            rows_v.at[b], out_hbm.at[pl.ds(base + j * _CHUNK, _CHUNK)], osem
        )
    for j in (_N_CHUNKS - 2, _N_CHUNKS - 1):
        pltpu.make_async_copy(
            rows_v.at[j % 2], out_hbm.at[pl.ds(base + j * _CHUNK, _CHUNK)], osem
        ).wait()


_PAD_ROWS = 10240


def _pad_body(xt_ref, o_ref):
    o_ref[:, :DIM] = jnp.transpose(xt_ref[...])


_pad_table = pl.pallas_call(
    _pad_body,
    grid=(pl.cdiv(VOCAB, _PAD_ROWS),),
    in_specs=[pl.BlockSpec((DIM, _PAD_ROWS), lambda i: (0, i))],
    out_specs=pl.BlockSpec((_PAD_ROWS, _DPAD), lambda i: (i, 0)),
    out_shape=jax.ShapeDtypeStruct((VOCAB, _DPAD), jnp.float32),
)

_CAT_ROWS = 2048
_N_CAT = HALF // _CAT_ROWS


def _concat_half_body(obj_ref, sub_ref, o_ref):
    o_ref[:DIM, :] = jnp.transpose(obj_ref[...])[:DIM, :]
    o_ref[DIM:, :] = jnp.transpose(sub_ref[...])[:DIM, :]


_concat_a = pl.pallas_call(
    _concat_half_body,
    grid=(_N_CAT,),
    in_specs=[
        pl.BlockSpec((_CAT_ROWS, _DPAD), lambda i: (i, 0)),
        pl.BlockSpec((_CAT_ROWS, _DPAD), lambda i: (i + _N_CAT, 0)),
    ],
    out_specs=pl.BlockSpec((2 * DIM, _CAT_ROWS), lambda i: (0, i)),
    out_shape=jax.ShapeDtypeStruct((2 * DIM, BATCH), jnp.float32),
)


def _concat_b_body(obj_ref, sub_ref, acc_ref, o_ref):
    del acc_ref
    o_ref[:DIM, :] = jnp.transpose(obj_ref[...])[:DIM, :]
    o_ref[DIM:, :] = jnp.transpose(sub_ref[...])[:DIM, :]


_concat_b = pl.pallas_call(
    _concat_b_body,
    grid=(_N_CAT,),
    in_specs=[
        pl.BlockSpec((_CAT_ROWS, _DPAD), lambda i: (i, 0)),
        pl.BlockSpec((_CAT_ROWS, _DPAD), lambda i: (i + _N_CAT, 0)),
        pl.BlockSpec(memory_space=pl.ANY),
    ],
    out_specs=pl.BlockSpec((2 * DIM, _CAT_ROWS), lambda i: (0, i + _N_CAT)),
    out_shape=jax.ShapeDtypeStruct((2 * DIM, BATCH), jnp.float32),
    input_output_aliases={2: 0},
)


def kernel(word_embs, obj_category, sub_category):
    obj = obj_category.astype(jnp.int32)
    sub = sub_category.astype(jnp.int32)
    idx_a = jnp.concatenate([obj[:HALF], sub[:HALF]]).reshape(
        N_ROWS // _CHUNK, _CHUNK
    )
    idx_b = jnp.concatenate([obj[HALF:], sub[HALF:]]).reshape(
        N_ROWS // _CHUNK, _CHUNK
    )
    table_p = _pad_table(word_embs.T)
    rows_a = _gather_rows(table_p, idx_a)
    rows_b = _gather_rows(table_p, idx_b)
    out_a = _concat_a(rows_a, rows_a)
    out = _concat_b(rows_b, rows_b, out_a)
    return out.T
